# baseline (speedup 1.0000x reference)
import jax
import jax.numpy as jnp
from jax.experimental import pallas as pl
from jax.experimental.pallas import tpu as pltpu


def _body(t_ref, betas_ref, bp_ref, go_ref, tr_ref,
          b_out, bp_out, go_out, tr_out):
    b_out[...] = betas_ref[...]
    bp_out[...] = bp_ref[0]
    go_out[...] = go_ref[0]
    tr_out[...] = tr_ref[0]


@jax.jit
def _tc_lookup(time, betas, body_pose, global_orient, transl):
    grid_spec = pltpu.PrefetchScalarGridSpec(
        num_scalar_prefetch=1,
        grid=(1,),
        in_specs=[
            pl.BlockSpec((1, 10), lambda i, t: (0, 0)),
            pl.BlockSpec((1, 1, 69), lambda i, t: (t[0], 0, 0)),
            pl.BlockSpec((1, 1, 3), lambda i, t: (t[0], 0, 0)),
            pl.BlockSpec((1, 1, 3), lambda i, t: (t[0], 0, 0)),
        ],
        out_specs=[
            pl.BlockSpec((1, 10), lambda i, t: (0, 0)),
            pl.BlockSpec((1, 69), lambda i, t: (0, 0)),
            pl.BlockSpec((1, 3), lambda i, t: (0, 0)),
            pl.BlockSpec((1, 3), lambda i, t: (0, 0)),
        ],
    )
    return pl.pallas_call(
        _body,
        grid_spec=grid_spec,
        out_shape=(
            jax.ShapeDtypeStruct((1, 10), jnp.float32),
            jax.ShapeDtypeStruct((1, 69), jnp.float32),
            jax.ShapeDtypeStruct((1, 3), jnp.float32),
            jax.ShapeDtypeStruct((1, 3), jnp.float32),
        ),
    )(time, betas, body_pose, global_orient, transl)


def kernel(time, betas, body_pose, global_orient, transl):
    return _tc_lookup(
        time.astype(jnp.int32), betas, body_pose, global_orient, transl
    )


# speedup vs baseline: 2.0028x; 2.0028x over previous
import jax
import jax.numpy as jnp
from jax import lax
from jax.experimental import pallas as pl
from jax.experimental.pallas import tpu as pltpu


def _select(ref, t):
    x = ref[:, 0, :]
    mask = lax.broadcasted_iota(jnp.int32, x.shape, 1) == t
    col = jnp.sum(jnp.where(mask, x, 0.0), axis=1, keepdims=True)
    return lax.transpose(col, (1, 0))


def _body(t_ref, betas_ref, bp_ref, go_ref, tr_ref,
          b_out, bp_out, go_out, tr_out):
    t = t_ref[0]
    b_out[...] = betas_ref[...]
    bp_out[...] = _select(bp_ref, t)
    go_out[...] = _select(go_ref, t)
    tr_out[...] = _select(tr_ref, t)


@jax.jit
def _tc_lookup(time, betas, body_pose, global_orient, transl):
    bp_t = jnp.transpose(body_pose, (2, 1, 0))
    go_t = jnp.transpose(global_orient, (2, 1, 0))
    tr_t = jnp.transpose(transl, (2, 1, 0))
    return pl.pallas_call(
        _body,
        in_specs=[
            pl.BlockSpec(memory_space=pltpu.MemorySpace.SMEM),
            pl.BlockSpec(memory_space=pltpu.MemorySpace.VMEM),
            pl.BlockSpec(memory_space=pltpu.MemorySpace.VMEM),
            pl.BlockSpec(memory_space=pltpu.MemorySpace.VMEM),
            pl.BlockSpec(memory_space=pltpu.MemorySpace.VMEM),
        ],
        out_specs=[
            pl.BlockSpec(memory_space=pltpu.MemorySpace.VMEM),
            pl.BlockSpec(memory_space=pltpu.MemorySpace.VMEM),
            pl.BlockSpec(memory_space=pltpu.MemorySpace.VMEM),
            pl.BlockSpec(memory_space=pltpu.MemorySpace.VMEM),
        ],
        out_shape=(
            jax.ShapeDtypeStruct((1, 10), jnp.float32),
            jax.ShapeDtypeStruct((1, 69), jnp.float32),
            jax.ShapeDtypeStruct((1, 3), jnp.float32),
            jax.ShapeDtypeStruct((1, 3), jnp.float32),
        ),
    )(time, betas, bp_t, go_t, tr_t)


def kernel(time, betas, body_pose, global_orient, transl):
    return _tc_lookup(
        time.astype(jnp.int32), betas, body_pose, global_orient, transl
    )
